# baseline (speedup 1.0000x reference)
import jax
import jax.numpy as jnp
from jax import lax
from jax.experimental import pallas as pl
from jax.experimental.pallas import tpu as pltpu
from jax.experimental.pallas import tpu_sc as plsc

N = 10000
E = 320000
D_IN = 128
D_HID = 256
D_OUT = 128
NC = 2
NS = 16
CH = 128
NCHUNK = E // CH
N_PAD = N + 8
N_PAD1 = 10112
ROWS_BIG = 640
ROWS_LAST = N - 15 * ROWS_BIG


def _part_rows(sid, fn):
    @pl.when(sid < 15)
    def _():
        fn(sid * ROWS_BIG, ROWS_BIG)

    @pl.when(sid == 15)
    def _():
        fn(15 * ROWS_BIG, ROWS_LAST)

_MESH = plsc.VectorSubcoreMesh(core_axis_name="c", subcore_axis_name="s")
_PREC = lax.Precision.HIGHEST


BB = 26


def _deg_body(row_hbm, col_hbm, rc_out, dega_out, degb_out,
              row_v, col_v, col2_v, ones_v, zeros_v, acc_sh, sem):
    del sem
    cid = lax.axis_index("c")
    sid = lax.axis_index("s")
    w = cid * NS + sid

    for j in range(8):
        ones_v[pl.ds(16 * j, 16)] = jnp.full((16,), 1.0, jnp.float32)

    def _zfill(j, carry):
        zeros_v[pl.ds(16 * j, 16)] = jnp.zeros((16,), jnp.float32)
        return carry
    lax.fori_loop(0, 64, _zfill, 0)

    def _izero(k, carry):
        pltpu.sync_copy(zeros_v.at[pl.ds(0, 128)],
                        acc_sh.at[pl.ds(k * 128, 128)])
        return carry

    @pl.when(w < 15)
    def _():
        lax.fori_loop(w * 5, w * 5 + 5, _izero, 0)

    @pl.when(w == 15)
    def _():
        lax.fori_loop(75, 79, _izero, 0)

    plsc.subcore_barrier()

    def _chunk(k, off):
        for j in range(CH // 16):
            r = row_v[pl.ds(off + 16 * j, 16)]
            c = col_v[pl.ds(off + 16 * j, 16)]
            col2_v[pl.ds(16 * j, 16)] = jnp.where(r == c, N, c)
        pltpu.sync_copy(row_v.at[pl.ds(off, CH)], rc_out.at[k, 0])
        pltpu.sync_copy(col2_v, rc_out.at[k, 1])
        pltpu.sync_copy(ones_v, acc_sh.at[col2_v], add=True)

    def _block(b, carry):
        k0 = w * 78 + b * BB
        pltpu.sync_copy(row_hbm.at[pl.ds(k0 * CH, BB * CH)], row_v)
        pltpu.sync_copy(col_hbm.at[pl.ds(k0 * CH, BB * CH)], col_v)

        def _inner(i, c2):
            _chunk(k0 + i, i * CH)
            return c2
        lax.fori_loop(0, BB, _inner, 0)
        return carry
    lax.fori_loop(0, 78 // BB, _block, 0)

    @pl.when(w < NCHUNK - 32 * 78)
    def _():
        k = 32 * 78 + w
        pltpu.sync_copy(row_hbm.at[pl.ds(k * CH, CH)], row_v.at[pl.ds(0, CH)])
        pltpu.sync_copy(col_hbm.at[pl.ds(k * CH, CH)], col_v.at[pl.ds(0, CH)])
        _chunk(k, 0)

    plsc.subcore_barrier()

    @pl.when(jnp.logical_and(cid == 0, sid == 0))
    def _():
        pltpu.sync_copy(acc_sh, dega_out)

    @pl.when(jnp.logical_and(cid == 1, sid == 0))
    def _():
        pltpu.sync_copy(acc_sh, degb_out)


_deg_call = pl.kernel(
    _deg_body,
    out_type=[
        jax.ShapeDtypeStruct((NCHUNK, 2, CH), jnp.int32),
        jax.ShapeDtypeStruct((N_PAD1,), jnp.float32),
        jax.ShapeDtypeStruct((N_PAD1,), jnp.float32),
    ],
    mesh=_MESH,
    scratch_types=[
        pltpu.VMEM((BB * CH,), jnp.int32),
        pltpu.VMEM((BB * CH,), jnp.int32),
        pltpu.VMEM((CH,), jnp.int32),
        pltpu.VMEM((CH,), jnp.float32),
        pltpu.VMEM((1024,), jnp.float32),
        pltpu.VMEM_SHARED((N_PAD1,), jnp.float32),
        pltpu.SemaphoreType.DMA,
    ],
)


IB = 26


def _scatter_edges(h_hbm, rc_hbm, acc_sh,
                   rc_blk, rc0, rows0, rows1, sem0, sem1,
                   chunk_lo, nblocks, extra_pred, extra_chunk):
    def _gather(i, rows, sem):
        return pltpu.async_copy(h_hbm.at[rc_blk.at[i, 0]], rows, sem)

    def _block(b, carry):
        k0 = chunk_lo + b * IB
        pltpu.sync_copy(rc_hbm.at[pl.ds(k0, IB)], rc_blk)
        d0 = _gather(0, rows0, sem0)

        def _pair(p, c2):
            d1 = _gather(2 * p + 1, rows1, sem1)
            d0.wait()
            pltpu.sync_copy(rows0, acc_sh.at[rc_blk.at[2 * p, 1]], add=True)

            @pl.when(p < IB // 2 - 1)
            def _():
                _gather(2 * p + 2, rows0, sem0)
            d1.wait()
            pltpu.sync_copy(rows1, acc_sh.at[rc_blk.at[2 * p + 1, 1]],
                            add=True)
            return c2
        lax.fori_loop(0, IB // 2, _pair, 0)
        return carry
    lax.fori_loop(0, nblocks, _block, 0)

    @pl.when(extra_pred)
    def _():
        pltpu.sync_copy(rc_hbm.at[extra_chunk], rc0)
        pltpu.async_copy(h_hbm.at[rc0.at[0]], rows0, sem0).wait()
        pltpu.sync_copy(rows0, acc_sh.at[rc0.at[1]], add=True)


def _prop1_body(ha_hbm, hb_hbm, rc_hbm, acca_out, accb_out,
                rc_blk, rc0, rows0, rows1, acc_sh, sem0, sem1):
    cid = lax.axis_index("c")
    sid = lax.axis_index("s")
    lo = sid * 156
    extra_pred = sid < NCHUNK - 16 * 156
    extra_chunk = 16 * 156 + sid

    def _core(h_hbm, acc_out):
        _part_rows(sid, lambda r0, nr: pltpu.sync_copy(
            h_hbm.at[pl.ds(pl.multiple_of(r0, 8), nr)],
            acc_sh.at[pl.ds(pl.multiple_of(r0, 8), nr)]))
        plsc.subcore_barrier()
        _scatter_edges(h_hbm, rc_hbm, acc_sh, rc_blk, rc0, rows0, rows1,
                       sem0, sem1, lo, 6, extra_pred, extra_chunk)
        plsc.subcore_barrier()
        _part_rows(sid, lambda r0, nr: pltpu.sync_copy(
            acc_sh.at[pl.ds(pl.multiple_of(r0, 8), nr)],
            acc_out.at[pl.ds(pl.multiple_of(r0, 8), nr)]))

    @pl.when(cid == 0)
    def _():
        _core(ha_hbm, acca_out)

    @pl.when(cid == 1)
    def _():
        _core(hb_hbm, accb_out)


_prop1_call = pl.kernel(
    _prop1_body,
    out_type=[
        jax.ShapeDtypeStruct((N, 128), jnp.float32),
        jax.ShapeDtypeStruct((N, 128), jnp.float32),
    ],
    mesh=_MESH,
    scratch_types=[
        pltpu.VMEM((IB, 2, CH), jnp.int32),
        pltpu.VMEM((2, CH), jnp.int32),
        pltpu.VMEM((CH, 128), jnp.float32),
        pltpu.VMEM((CH, 128), jnp.float32),
        pltpu.VMEM_SHARED((N_PAD, 128), jnp.float32),
        pltpu.SemaphoreType.DMA,
        pltpu.SemaphoreType.DMA,
    ],
)


def _prop2_body(h_hbm, rc_hbm, acc0_out, acc1_out,
                rc_blk, rc0, rows0, rows1, acc_sh, sem0, sem1):
    cid = lax.axis_index("c")
    sid = lax.axis_index("s")
    _part_rows(sid, lambda r0, nr: pltpu.sync_copy(
        h_hbm.at[pl.ds(pl.multiple_of(r0, 8), nr)],
        acc_sh.at[pl.ds(pl.multiple_of(r0, 8), nr)]))
    plsc.subcore_barrier()
    lo = cid * 1250 + sid * 78
    extra_pred = sid < 2
    extra_chunk = cid * 1250 + 16 * 78 + sid
    _scatter_edges(h_hbm, rc_hbm, acc_sh, rc_blk, rc0, rows0, rows1,
                   sem0, sem1, lo, 3, extra_pred, extra_chunk)
    plsc.subcore_barrier()

    @pl.when(cid == 0)
    def _():
        _part_rows(sid, lambda r0, nr: pltpu.sync_copy(
            acc_sh.at[pl.ds(pl.multiple_of(r0, 8), nr)],
            acc0_out.at[pl.ds(pl.multiple_of(r0, 8), nr)]))

    @pl.when(cid == 1)
    def _():
        _part_rows(sid, lambda r0, nr: pltpu.sync_copy(
            acc_sh.at[pl.ds(pl.multiple_of(r0, 8), nr)],
            acc1_out.at[pl.ds(pl.multiple_of(r0, 8), nr)]))


_prop2_call = pl.kernel(
    _prop2_body,
    out_type=[
        jax.ShapeDtypeStruct((N, 128), jnp.float32),
        jax.ShapeDtypeStruct((N, 128), jnp.float32),
    ],
    mesh=_MESH,
    scratch_types=[
        pltpu.VMEM((IB, 2, CH), jnp.int32),
        pltpu.VMEM((2, CH), jnp.int32),
        pltpu.VMEM((CH, 128), jnp.float32),
        pltpu.VMEM((CH, 128), jnp.float32),
        pltpu.VMEM_SHARED((N_PAD, 128), jnp.float32),
        pltpu.SemaphoreType.DMA,
        pltpu.SemaphoreType.DMA,
    ],
)


BLK = 5000


def _tcA1_body(x_ref, w1_ref, b1_ref, h_ref):
    h = lax.dot_general(x_ref[...], w1_ref[...], (((1,), (1,)), ((), ())),
                        precision=_PREC, preferred_element_type=jnp.float32)
    h_ref[...] = h + b1_ref[...]


def _tcA1(x, W1, b1r):
    return pl.pallas_call(
        _tcA1_body,
        grid=(N // BLK,),
        in_specs=[
            pl.BlockSpec((BLK, D_IN), lambda i: (i, 0)),
            pl.BlockSpec((D_HID, D_IN), lambda i: (0, 0)),
            pl.BlockSpec((1, D_HID), lambda i: (0, 0)),
        ],
        out_specs=pl.BlockSpec((BLK, D_HID), lambda i: (i, 0)),
        out_shape=jax.ShapeDtypeStruct((N, D_HID), jnp.float32),
    )(x, W1, b1r)


def _tcA2_body(h_ref, dega_ref, degb_ref, ha_ref, hb_ref, dis_ref):
    deg = dega_ref[...] + degb_ref[...] + 1.0
    dis = lax.rsqrt(deg)
    hs = h_ref[...] * dis
    ha_ref[...] = hs[:, :128]
    hb_ref[...] = hs[:, 128:]
    dis_ref[...] = dis


def _tcA2(h1, dega2, degb2):
    return pl.pallas_call(
        _tcA2_body,
        grid=(N // BLK,),
        in_specs=[
            pl.BlockSpec((BLK, D_HID), lambda i: (i, 0)),
            pl.BlockSpec((BLK, 1), lambda i: (i, 0)),
            pl.BlockSpec((BLK, 1), lambda i: (i, 0)),
        ],
        out_specs=[
            pl.BlockSpec((BLK, 128), lambda i: (i, 0)),
            pl.BlockSpec((BLK, 128), lambda i: (i, 0)),
            pl.BlockSpec((BLK, 1), lambda i: (i, 0)),
        ],
        out_shape=[
            jax.ShapeDtypeStruct((N, 128), jnp.float32),
            jax.ShapeDtypeStruct((N, 128), jnp.float32),
            jax.ShapeDtypeStruct((N, 1), jnp.float32),
        ],
    )(h1, dega2, degb2)


def _tcB_body(pa_ref, pb_ref, dis_ref, w2a_ref, w2b_ref, b2_ref, out_ref):
    d = dis_ref[...]
    ra = jnp.maximum(pa_ref[...] * d, 0.0)
    rb = jnp.maximum(pb_ref[...] * d, 0.0)
    h2 = (lax.dot_general(ra, w2a_ref[...], (((1,), (1,)), ((), ())),
                          precision=_PREC, preferred_element_type=jnp.float32)
          + lax.dot_general(rb, w2b_ref[...], (((1,), (1,)), ((), ())),
                            precision=_PREC, preferred_element_type=jnp.float32)
          + b2_ref[...])
    out_ref[...] = h2 * d


def _tcB(acca, accb, dis, W2a, W2b, b2r):
    return pl.pallas_call(
        _tcB_body,
        grid=(N // BLK,),
        in_specs=[
            pl.BlockSpec((BLK, 128), lambda i: (i, 0)),
            pl.BlockSpec((BLK, 128), lambda i: (i, 0)),
            pl.BlockSpec((BLK, 1), lambda i: (i, 0)),
            pl.BlockSpec((D_OUT, 128), lambda i: (0, 0)),
            pl.BlockSpec((D_OUT, 128), lambda i: (0, 0)),
            pl.BlockSpec((1, D_OUT), lambda i: (0, 0)),
        ],
        out_specs=pl.BlockSpec((BLK, D_OUT), lambda i: (i, 0)),
        out_shape=jax.ShapeDtypeStruct((N, D_OUT), jnp.float32),
    )(acca, accb, dis, W2a, W2b, b2r)


def _tcC_body(a0_ref, a1_ref, h2s_ref, dis_ref, out_ref):
    p = (a0_ref[...] + a1_ref[...] - h2s_ref[...]) * dis_ref[...]
    m = jnp.max(p, axis=1, keepdims=True)
    e = jnp.exp(p - m)
    s = jnp.sum(e, axis=1, keepdims=True)
    out_ref[...] = p - m - jnp.log(s)


def _tcC(a0, a1, h2s, dis):
    return pl.pallas_call(
        _tcC_body,
        grid=(N // BLK,),
        in_specs=[
            pl.BlockSpec((BLK, D_OUT), lambda i: (i, 0)),
            pl.BlockSpec((BLK, D_OUT), lambda i: (i, 0)),
            pl.BlockSpec((BLK, D_OUT), lambda i: (i, 0)),
            pl.BlockSpec((BLK, 1), lambda i: (i, 0)),
        ],
        out_specs=pl.BlockSpec((BLK, D_OUT), lambda i: (i, 0)),
        out_shape=jax.ShapeDtypeStruct((N, D_OUT), jnp.float32),
    )(a0, a1, h2s, dis)


def kernel(x, edge_index, W1, b1, W2, b2):
    row = edge_index[0]
    col = edge_index[1]
    rc2, dega, degb = _deg_call(row, col)
    h1 = _tcA1(x, W1, b1.reshape(1, D_HID))
    ha, hb, dis = _tcA2(h1, dega[:N].reshape(N, 1), degb[:N].reshape(N, 1))
    acca, accb = _prop1_call(ha, hb, rc2)
    h2s = _tcB(acca, accb, dis, W2[:, :128], W2[:, 128:], b2.reshape(1, D_OUT))
    a0, a1 = _prop2_call(h2s, rc2)
    return _tcC(a0, a1, h2s, dis)


# speedup vs baseline: 1.0253x; 1.0253x over previous
import jax
import jax.numpy as jnp
from jax import lax
from jax.experimental import pallas as pl
from jax.experimental.pallas import tpu as pltpu
from jax.experimental.pallas import tpu_sc as plsc

N = 10000
E = 320000
D_IN = 128
D_HID = 256
D_OUT = 128
NC = 2
NS = 16
CH = 128
NCHUNK = E // CH
N_PAD = N + 8
N_PAD1 = 10112
ROWS_BIG = 640
ROWS_LAST = N - 15 * ROWS_BIG


def _part_rows(sid, fn):
    @pl.when(sid < 15)
    def _():
        fn(sid * ROWS_BIG, ROWS_BIG)

    @pl.when(sid == 15)
    def _():
        fn(15 * ROWS_BIG, ROWS_LAST)

_MESH = plsc.VectorSubcoreMesh(core_axis_name="c", subcore_axis_name="s")
_PREC = lax.Precision.DEFAULT


BB = 26


def _deg_body(row_hbm, col_hbm, rc_out, dega_out, degb_out,
              row_v, col_v, col2_v, ones_v, zeros_v, acc_sh, sem):
    del sem
    cid = lax.axis_index("c")
    sid = lax.axis_index("s")
    w = cid * NS + sid

    for j in range(8):
        ones_v[pl.ds(16 * j, 16)] = jnp.full((16,), 1.0, jnp.float32)

    def _zfill(j, carry):
        zeros_v[pl.ds(16 * j, 16)] = jnp.zeros((16,), jnp.float32)
        return carry
    lax.fori_loop(0, 64, _zfill, 0)

    def _izero(k, carry):
        pltpu.sync_copy(zeros_v.at[pl.ds(0, 128)],
                        acc_sh.at[pl.ds(k * 128, 128)])
        return carry

    @pl.when(w < 15)
    def _():
        lax.fori_loop(w * 5, w * 5 + 5, _izero, 0)

    @pl.when(w == 15)
    def _():
        lax.fori_loop(75, 79, _izero, 0)

    plsc.subcore_barrier()

    def _chunk(k, off):
        for j in range(CH // 16):
            r = row_v[pl.ds(off + 16 * j, 16)]
            c = col_v[pl.ds(off + 16 * j, 16)]
            col2_v[pl.ds(16 * j, 16)] = jnp.where(r == c, N, c)
        pltpu.sync_copy(row_v.at[pl.ds(off, CH)], rc_out.at[k, 0])
        pltpu.sync_copy(col2_v, rc_out.at[k, 1])
        pltpu.sync_copy(ones_v, acc_sh.at[col2_v], add=True)

    def _block(b, carry):
        k0 = w * 78 + b * BB
        pltpu.sync_copy(row_hbm.at[pl.ds(k0 * CH, BB * CH)], row_v)
        pltpu.sync_copy(col_hbm.at[pl.ds(k0 * CH, BB * CH)], col_v)

        def _inner(i, c2):
            _chunk(k0 + i, i * CH)
            return c2
        lax.fori_loop(0, BB, _inner, 0)
        return carry
    lax.fori_loop(0, 78 // BB, _block, 0)

    @pl.when(w < NCHUNK - 32 * 78)
    def _():
        k = 32 * 78 + w
        pltpu.sync_copy(row_hbm.at[pl.ds(k * CH, CH)], row_v.at[pl.ds(0, CH)])
        pltpu.sync_copy(col_hbm.at[pl.ds(k * CH, CH)], col_v.at[pl.ds(0, CH)])
        _chunk(k, 0)

    plsc.subcore_barrier()

    @pl.when(jnp.logical_and(cid == 0, sid == 0))
    def _():
        pltpu.sync_copy(acc_sh, dega_out)

    @pl.when(jnp.logical_and(cid == 1, sid == 0))
    def _():
        pltpu.sync_copy(acc_sh, degb_out)


_deg_call = pl.kernel(
    _deg_body,
    out_type=[
        jax.ShapeDtypeStruct((NCHUNK, 2, CH), jnp.int32),
        jax.ShapeDtypeStruct((N_PAD1,), jnp.float32),
        jax.ShapeDtypeStruct((N_PAD1,), jnp.float32),
    ],
    mesh=_MESH,
    scratch_types=[
        pltpu.VMEM((BB * CH,), jnp.int32),
        pltpu.VMEM((BB * CH,), jnp.int32),
        pltpu.VMEM((CH,), jnp.int32),
        pltpu.VMEM((CH,), jnp.float32),
        pltpu.VMEM((1024,), jnp.float32),
        pltpu.VMEM_SHARED((N_PAD1,), jnp.float32),
        pltpu.SemaphoreType.DMA,
    ],
)


IB = 26


def _scatter_edges(h_hbm, rc_hbm, acc_sh,
                   rc_blk, rc0, rows0, rows1, sem0, sem1,
                   chunk_lo, nblocks, extra_pred, extra_chunk):
    def _gather(i, rows, sem):
        return pltpu.async_copy(h_hbm.at[rc_blk.at[i, 0]], rows, sem)

    def _block(b, carry):
        k0 = chunk_lo + b * IB
        pltpu.sync_copy(rc_hbm.at[pl.ds(k0, IB)], rc_blk)
        d0 = _gather(0, rows0, sem0)

        def _pair(p, c2):
            d1 = _gather(2 * p + 1, rows1, sem1)
            d0.wait()
            pltpu.sync_copy(rows0, acc_sh.at[rc_blk.at[2 * p, 1]], add=True)

            @pl.when(p < IB // 2 - 1)
            def _():
                _gather(2 * p + 2, rows0, sem0)
            d1.wait()
            pltpu.sync_copy(rows1, acc_sh.at[rc_blk.at[2 * p + 1, 1]],
                            add=True)
            return c2
        lax.fori_loop(0, IB // 2, _pair, 0)
        return carry
    lax.fori_loop(0, nblocks, _block, 0)

    @pl.when(extra_pred)
    def _():
        pltpu.sync_copy(rc_hbm.at[extra_chunk], rc0)
        pltpu.async_copy(h_hbm.at[rc0.at[0]], rows0, sem0).wait()
        pltpu.sync_copy(rows0, acc_sh.at[rc0.at[1]], add=True)


def _prop1_body(ha_hbm, hb_hbm, rc_hbm, acca_out, accb_out,
                rc_blk, rc0, rows0, rows1, acc_sh, sem0, sem1):
    cid = lax.axis_index("c")
    sid = lax.axis_index("s")
    lo = sid * 156
    extra_pred = sid < NCHUNK - 16 * 156
    extra_chunk = 16 * 156 + sid

    def _core(h_hbm, acc_out):
        _part_rows(sid, lambda r0, nr: pltpu.sync_copy(
            h_hbm.at[pl.ds(pl.multiple_of(r0, 8), nr)],
            acc_sh.at[pl.ds(pl.multiple_of(r0, 8), nr)]))
        plsc.subcore_barrier()
        _scatter_edges(h_hbm, rc_hbm, acc_sh, rc_blk, rc0, rows0, rows1,
                       sem0, sem1, lo, 6, extra_pred, extra_chunk)
        plsc.subcore_barrier()
        _part_rows(sid, lambda r0, nr: pltpu.sync_copy(
            acc_sh.at[pl.ds(pl.multiple_of(r0, 8), nr)],
            acc_out.at[pl.ds(pl.multiple_of(r0, 8), nr)]))

    @pl.when(cid == 0)
    def _():
        _core(ha_hbm, acca_out)

    @pl.when(cid == 1)
    def _():
        _core(hb_hbm, accb_out)


_prop1_call = pl.kernel(
    _prop1_body,
    out_type=[
        jax.ShapeDtypeStruct((N, 128), jnp.float32),
        jax.ShapeDtypeStruct((N, 128), jnp.float32),
    ],
    mesh=_MESH,
    scratch_types=[
        pltpu.VMEM((IB, 2, CH), jnp.int32),
        pltpu.VMEM((2, CH), jnp.int32),
        pltpu.VMEM((CH, 128), jnp.float32),
        pltpu.VMEM((CH, 128), jnp.float32),
        pltpu.VMEM_SHARED((N_PAD, 128), jnp.float32),
        pltpu.SemaphoreType.DMA,
        pltpu.SemaphoreType.DMA,
    ],
)


def _prop2_body(h_hbm, rc_hbm, acc0_out, acc1_out,
                rc_blk, rc0, rows0, rows1, acc_sh, sem0, sem1):
    cid = lax.axis_index("c")
    sid = lax.axis_index("s")
    _part_rows(sid, lambda r0, nr: pltpu.sync_copy(
        h_hbm.at[pl.ds(pl.multiple_of(r0, 8), nr)],
        acc_sh.at[pl.ds(pl.multiple_of(r0, 8), nr)]))
    plsc.subcore_barrier()
    lo = cid * 1250 + sid * 78
    extra_pred = sid < 2
    extra_chunk = cid * 1250 + 16 * 78 + sid
    _scatter_edges(h_hbm, rc_hbm, acc_sh, rc_blk, rc0, rows0, rows1,
                   sem0, sem1, lo, 3, extra_pred, extra_chunk)
    plsc.subcore_barrier()

    @pl.when(cid == 0)
    def _():
        _part_rows(sid, lambda r0, nr: pltpu.sync_copy(
            acc_sh.at[pl.ds(pl.multiple_of(r0, 8), nr)],
            acc0_out.at[pl.ds(pl.multiple_of(r0, 8), nr)]))

    @pl.when(cid == 1)
    def _():
        _part_rows(sid, lambda r0, nr: pltpu.sync_copy(
            acc_sh.at[pl.ds(pl.multiple_of(r0, 8), nr)],
            acc1_out.at[pl.ds(pl.multiple_of(r0, 8), nr)]))


_prop2_call = pl.kernel(
    _prop2_body,
    out_type=[
        jax.ShapeDtypeStruct((N, 128), jnp.float32),
        jax.ShapeDtypeStruct((N, 128), jnp.float32),
    ],
    mesh=_MESH,
    scratch_types=[
        pltpu.VMEM((IB, 2, CH), jnp.int32),
        pltpu.VMEM((2, CH), jnp.int32),
        pltpu.VMEM((CH, 128), jnp.float32),
        pltpu.VMEM((CH, 128), jnp.float32),
        pltpu.VMEM_SHARED((N_PAD, 128), jnp.float32),
        pltpu.SemaphoreType.DMA,
        pltpu.SemaphoreType.DMA,
    ],
)


BLK = 2000


def _tcA1_body(x_ref, w1_ref, b1_ref, h_ref):
    h = lax.dot_general(x_ref[...], w1_ref[...], (((1,), (1,)), ((), ())),
                        precision=_PREC, preferred_element_type=jnp.float32)
    h_ref[...] = h + b1_ref[...]


def _tcA1(x, W1, b1r):
    return pl.pallas_call(
        _tcA1_body,
        grid=(N // BLK,),
        in_specs=[
            pl.BlockSpec((BLK, D_IN), lambda i: (i, 0)),
            pl.BlockSpec((D_HID, D_IN), lambda i: (0, 0)),
            pl.BlockSpec((1, D_HID), lambda i: (0, 0)),
        ],
        out_specs=pl.BlockSpec((BLK, D_HID), lambda i: (i, 0)),
        out_shape=jax.ShapeDtypeStruct((N, D_HID), jnp.float32),
    )(x, W1, b1r)


def _tcA2_body(h_ref, dega_ref, degb_ref, ha_ref, hb_ref, dis_ref):
    deg = dega_ref[...] + degb_ref[...] + 1.0
    dis = lax.rsqrt(deg)
    hs = h_ref[...] * dis
    ha_ref[...] = hs[:, :128]
    hb_ref[...] = hs[:, 128:]
    dis_ref[...] = dis


def _tcA2(h1, dega2, degb2):
    return pl.pallas_call(
        _tcA2_body,
        grid=(N // BLK,),
        in_specs=[
            pl.BlockSpec((BLK, D_HID), lambda i: (i, 0)),
            pl.BlockSpec((BLK, 1), lambda i: (i, 0)),
            pl.BlockSpec((BLK, 1), lambda i: (i, 0)),
        ],
        out_specs=[
            pl.BlockSpec((BLK, 128), lambda i: (i, 0)),
            pl.BlockSpec((BLK, 128), lambda i: (i, 0)),
            pl.BlockSpec((BLK, 1), lambda i: (i, 0)),
        ],
        out_shape=[
            jax.ShapeDtypeStruct((N, 128), jnp.float32),
            jax.ShapeDtypeStruct((N, 128), jnp.float32),
            jax.ShapeDtypeStruct((N, 1), jnp.float32),
        ],
    )(h1, dega2, degb2)


def _tcB_body(pa_ref, pb_ref, dis_ref, w2a_ref, w2b_ref, b2_ref, out_ref):
    d = dis_ref[...]
    ra = jnp.maximum(pa_ref[...] * d, 0.0)
    rb = jnp.maximum(pb_ref[...] * d, 0.0)
    h2 = (lax.dot_general(ra, w2a_ref[...], (((1,), (1,)), ((), ())),
                          precision=_PREC, preferred_element_type=jnp.float32)
          + lax.dot_general(rb, w2b_ref[...], (((1,), (1,)), ((), ())),
                            precision=_PREC, preferred_element_type=jnp.float32)
          + b2_ref[...])
    out_ref[...] = h2 * d


def _tcB(acca, accb, dis, W2a, W2b, b2r):
    return pl.pallas_call(
        _tcB_body,
        grid=(N // BLK,),
        in_specs=[
            pl.BlockSpec((BLK, 128), lambda i: (i, 0)),
            pl.BlockSpec((BLK, 128), lambda i: (i, 0)),
            pl.BlockSpec((BLK, 1), lambda i: (i, 0)),
            pl.BlockSpec((D_OUT, 128), lambda i: (0, 0)),
            pl.BlockSpec((D_OUT, 128), lambda i: (0, 0)),
            pl.BlockSpec((1, D_OUT), lambda i: (0, 0)),
        ],
        out_specs=pl.BlockSpec((BLK, D_OUT), lambda i: (i, 0)),
        out_shape=jax.ShapeDtypeStruct((N, D_OUT), jnp.float32),
    )(acca, accb, dis, W2a, W2b, b2r)


def _tcC_body(a0_ref, a1_ref, h2s_ref, dis_ref, out_ref):
    p = (a0_ref[...] + a1_ref[...] - h2s_ref[...]) * dis_ref[...]
    m = jnp.max(p, axis=1, keepdims=True)
    e = jnp.exp(p - m)
    s = jnp.sum(e, axis=1, keepdims=True)
    out_ref[...] = p - m - jnp.log(s)


def _tcC(a0, a1, h2s, dis):
    return pl.pallas_call(
        _tcC_body,
        grid=(N // BLK,),
        in_specs=[
            pl.BlockSpec((BLK, D_OUT), lambda i: (i, 0)),
            pl.BlockSpec((BLK, D_OUT), lambda i: (i, 0)),
            pl.BlockSpec((BLK, D_OUT), lambda i: (i, 0)),
            pl.BlockSpec((BLK, 1), lambda i: (i, 0)),
        ],
        out_specs=pl.BlockSpec((BLK, D_OUT), lambda i: (i, 0)),
        out_shape=jax.ShapeDtypeStruct((N, D_OUT), jnp.float32),
    )(a0, a1, h2s, dis)


def kernel(x, edge_index, W1, b1, W2, b2):
    row = edge_index[0]
    col = edge_index[1]
    rc2, dega, degb = _deg_call(row, col)
    h1 = _tcA1(x, W1, b1.reshape(1, D_HID))
    ha, hb, dis = _tcA2(h1, dega[:N].reshape(N, 1), degb[:N].reshape(N, 1))
    acca, accb = _prop1_call(ha, hb, rc2)
    h2s = _tcB(acca, accb, dis, W2[:, :128], W2[:, 128:], b2.reshape(1, D_OUT))
    a0, a1 = _prop2_call(h2s, rc2)
    return _tcC(a0, a1, h2s, dis)


# speedup vs baseline: 1.0404x; 1.0146x over previous
import jax
import jax.numpy as jnp
from jax import lax
from jax.experimental import pallas as pl
from jax.experimental.pallas import tpu as pltpu
from jax.experimental.pallas import tpu_sc as plsc

N = 10000
E = 320000
D_IN = 128
D_HID = 256
D_OUT = 128
NC = 2
NS = 16
CH = 128
NCHUNK = E // CH
N_PAD = N + 8
N_PAD1 = 10112
ROWS_BIG = 640
ROWS_LAST = N - 15 * ROWS_BIG


def _part_rows(sid, fn):
    @pl.when(sid < 15)
    def _():
        fn(sid * ROWS_BIG, ROWS_BIG)

    @pl.when(sid == 15)
    def _():
        fn(15 * ROWS_BIG, ROWS_LAST)

_MESH = plsc.VectorSubcoreMesh(core_axis_name="c", subcore_axis_name="s")
_PREC = lax.Precision.DEFAULT


BB = 26


def _deg_body(row_hbm, col_hbm, rc_out, dega_out, degb_out,
              row_v, col_v, col2_v, ones_v, zeros_v, acc_sh, sem):
    del sem
    cid = lax.axis_index("c")
    sid = lax.axis_index("s")
    w = cid * NS + sid

    for j in range(8):
        ones_v[pl.ds(16 * j, 16)] = jnp.full((16,), 1.0, jnp.float32)

    def _zfill(j, carry):
        zeros_v[pl.ds(16 * j, 16)] = jnp.zeros((16,), jnp.float32)
        return carry
    lax.fori_loop(0, 64, _zfill, 0)

    def _izero(k, carry):
        pltpu.sync_copy(zeros_v.at[pl.ds(0, 128)],
                        acc_sh.at[pl.ds(k * 128, 128)])
        return carry

    @pl.when(w < 15)
    def _():
        lax.fori_loop(w * 5, w * 5 + 5, _izero, 0)

    @pl.when(w == 15)
    def _():
        lax.fori_loop(75, 79, _izero, 0)

    plsc.subcore_barrier()

    def _chunk(k, off):
        for j in range(CH // 16):
            r = row_v[pl.ds(off + 16 * j, 16)]
            c = col_v[pl.ds(off + 16 * j, 16)]
            col2_v[pl.ds(16 * j, 16)] = jnp.where(r == c, N, c)
        pltpu.sync_copy(row_v.at[pl.ds(off, CH)], rc_out.at[k, 0])
        pltpu.sync_copy(col2_v, rc_out.at[k, 1])
        pltpu.sync_copy(ones_v, acc_sh.at[col2_v], add=True)

    def _block(b, carry):
        k0 = w * 78 + b * BB
        pltpu.sync_copy(row_hbm.at[pl.ds(k0 * CH, BB * CH)], row_v)
        pltpu.sync_copy(col_hbm.at[pl.ds(k0 * CH, BB * CH)], col_v)

        def _inner(i, c2):
            _chunk(k0 + i, i * CH)
            return c2
        lax.fori_loop(0, BB, _inner, 0)
        return carry
    lax.fori_loop(0, 78 // BB, _block, 0)

    @pl.when(w < NCHUNK - 32 * 78)
    def _():
        k = 32 * 78 + w
        pltpu.sync_copy(row_hbm.at[pl.ds(k * CH, CH)], row_v.at[pl.ds(0, CH)])
        pltpu.sync_copy(col_hbm.at[pl.ds(k * CH, CH)], col_v.at[pl.ds(0, CH)])
        _chunk(k, 0)

    plsc.subcore_barrier()

    @pl.when(jnp.logical_and(cid == 0, sid == 0))
    def _():
        pltpu.sync_copy(acc_sh, dega_out)

    @pl.when(jnp.logical_and(cid == 1, sid == 0))
    def _():
        pltpu.sync_copy(acc_sh, degb_out)


_deg_call = pl.kernel(
    _deg_body,
    out_type=[
        jax.ShapeDtypeStruct((NCHUNK, 2, CH), jnp.int32),
        jax.ShapeDtypeStruct((N_PAD1,), jnp.float32),
        jax.ShapeDtypeStruct((N_PAD1,), jnp.float32),
    ],
    mesh=_MESH,
    scratch_types=[
        pltpu.VMEM((BB * CH,), jnp.int32),
        pltpu.VMEM((BB * CH,), jnp.int32),
        pltpu.VMEM((CH,), jnp.int32),
        pltpu.VMEM((CH,), jnp.float32),
        pltpu.VMEM((1024,), jnp.float32),
        pltpu.VMEM_SHARED((N_PAD1,), jnp.float32),
        pltpu.SemaphoreType.DMA,
    ],
)


IB = 26


def _scatter_edges(h_hbm, rc_hbm, acc_sh,
                   blkA, blkB, rc0, rows0, rows1, sem0, sem1, semb,
                   chunk_lo, nblocks, extra_pred, extra_chunk, init_fn):
    def _gather(blk, i, rows, sem):
        return pltpu.async_copy(h_hbm.at[blk.at[i, 0]], rows, sem)

    def _load_blk(b, blk):
        return pltpu.async_copy(rc_hbm.at[pl.ds(chunk_lo + b * IB, IB)],
                                blk, semb)

    def _pairs(blk):
        d0 = _gather(blk, 0, rows0, sem0)

        def _pair(p, c2):
            d1 = _gather(blk, 2 * p + 1, rows1, sem1)
            d0.wait()
            pltpu.sync_copy(rows0, acc_sh.at[blk.at[2 * p, 1]], add=True)

            @pl.when(p < IB // 2 - 1)
            def _():
                _gather(blk, 2 * p + 2, rows0, sem0)
            d1.wait()
            pltpu.sync_copy(rows1, acc_sh.at[blk.at[2 * p + 1, 1]], add=True)
            return c2
        lax.fori_loop(0, IB // 2, _pair, 0)

    bufs = (blkA, blkB)
    pending = {0: _load_blk(0, blkA)}
    init_fn()
    for b in range(nblocks):
        pending.pop(b).wait()
        if b + 1 < nblocks:
            pending[b + 1] = _load_blk(b + 1, bufs[(b + 1) % 2])
        _pairs(bufs[b % 2])

    @pl.when(extra_pred)
    def _():
        pltpu.sync_copy(rc_hbm.at[extra_chunk], rc0)
        pltpu.async_copy(h_hbm.at[rc0.at[0]], rows0, sem0).wait()
        pltpu.sync_copy(rows0, acc_sh.at[rc0.at[1]], add=True)


def _prop1_body(ha_hbm, hb_hbm, rc_hbm, acca_out, accb_out,
                blkA, blkB, rc0, rows0, rows1, acc_sh, sem0, sem1, semb):
    cid = lax.axis_index("c")
    sid = lax.axis_index("s")
    lo = sid * 156
    extra_pred = sid < NCHUNK - 16 * 156
    extra_chunk = 16 * 156 + sid

    def _core(h_hbm, acc_out):
        def _init():
            _part_rows(sid, lambda r0, nr: pltpu.sync_copy(
                h_hbm.at[pl.ds(pl.multiple_of(r0, 8), nr)],
                acc_sh.at[pl.ds(pl.multiple_of(r0, 8), nr)]))
            plsc.subcore_barrier()
        _scatter_edges(h_hbm, rc_hbm, acc_sh, blkA, blkB, rc0, rows0, rows1,
                       sem0, sem1, semb, lo, 6, extra_pred, extra_chunk,
                       _init)
        plsc.subcore_barrier()
        _part_rows(sid, lambda r0, nr: pltpu.sync_copy(
            acc_sh.at[pl.ds(pl.multiple_of(r0, 8), nr)],
            acc_out.at[pl.ds(pl.multiple_of(r0, 8), nr)]))

    @pl.when(cid == 0)
    def _():
        _core(ha_hbm, acca_out)

    @pl.when(cid == 1)
    def _():
        _core(hb_hbm, accb_out)


_prop1_call = pl.kernel(
    _prop1_body,
    out_type=[
        jax.ShapeDtypeStruct((N, 128), jnp.float32),
        jax.ShapeDtypeStruct((N, 128), jnp.float32),
    ],
    mesh=_MESH,
    scratch_types=[
        pltpu.VMEM((IB, 2, CH), jnp.int32),
        pltpu.VMEM((IB, 2, CH), jnp.int32),
        pltpu.VMEM((2, CH), jnp.int32),
        pltpu.VMEM((CH, 128), jnp.float32),
        pltpu.VMEM((CH, 128), jnp.float32),
        pltpu.VMEM_SHARED((N_PAD, 128), jnp.float32),
        pltpu.SemaphoreType.DMA,
        pltpu.SemaphoreType.DMA,
        pltpu.SemaphoreType.DMA,
    ],
)


def _prop2_body(h_hbm, rc_hbm, acc0_out, acc1_out,
                blkA, blkB, rc0, rows0, rows1, acc_sh, sem0, sem1, semb):
    cid = lax.axis_index("c")
    sid = lax.axis_index("s")
    def _init():
        _part_rows(sid, lambda r0, nr: pltpu.sync_copy(
            h_hbm.at[pl.ds(pl.multiple_of(r0, 8), nr)],
            acc_sh.at[pl.ds(pl.multiple_of(r0, 8), nr)]))
        plsc.subcore_barrier()
    lo = cid * 1250 + sid * 78
    extra_pred = sid < 2
    extra_chunk = cid * 1250 + 16 * 78 + sid
    _scatter_edges(h_hbm, rc_hbm, acc_sh, blkA, blkB, rc0, rows0, rows1,
                   sem0, sem1, semb, lo, 3, extra_pred, extra_chunk, _init)
    plsc.subcore_barrier()

    @pl.when(cid == 0)
    def _():
        _part_rows(sid, lambda r0, nr: pltpu.sync_copy(
            acc_sh.at[pl.ds(pl.multiple_of(r0, 8), nr)],
            acc0_out.at[pl.ds(pl.multiple_of(r0, 8), nr)]))

    @pl.when(cid == 1)
    def _():
        _part_rows(sid, lambda r0, nr: pltpu.sync_copy(
            acc_sh.at[pl.ds(pl.multiple_of(r0, 8), nr)],
            acc1_out.at[pl.ds(pl.multiple_of(r0, 8), nr)]))


_prop2_call = pl.kernel(
    _prop2_body,
    out_type=[
        jax.ShapeDtypeStruct((N, 128), jnp.float32),
        jax.ShapeDtypeStruct((N, 128), jnp.float32),
    ],
    mesh=_MESH,
    scratch_types=[
        pltpu.VMEM((IB, 2, CH), jnp.int32),
        pltpu.VMEM((IB, 2, CH), jnp.int32),
        pltpu.VMEM((2, CH), jnp.int32),
        pltpu.VMEM((CH, 128), jnp.float32),
        pltpu.VMEM((CH, 128), jnp.float32),
        pltpu.VMEM_SHARED((N_PAD, 128), jnp.float32),
        pltpu.SemaphoreType.DMA,
        pltpu.SemaphoreType.DMA,
        pltpu.SemaphoreType.DMA,
    ],
)


BLK = 2000


def _tcA1_body(x_ref, w1_ref, b1_ref, h_ref):
    h = lax.dot_general(x_ref[...], w1_ref[...], (((1,), (1,)), ((), ())),
                        precision=_PREC, preferred_element_type=jnp.float32)
    h_ref[...] = h + b1_ref[...]


def _tcA1(x, W1, b1r):
    return pl.pallas_call(
        _tcA1_body,
        grid=(N // BLK,),
        in_specs=[
            pl.BlockSpec((BLK, D_IN), lambda i: (i, 0)),
            pl.BlockSpec((D_HID, D_IN), lambda i: (0, 0)),
            pl.BlockSpec((1, D_HID), lambda i: (0, 0)),
        ],
        out_specs=pl.BlockSpec((BLK, D_HID), lambda i: (i, 0)),
        out_shape=jax.ShapeDtypeStruct((N, D_HID), jnp.float32),
    )(x, W1, b1r)


def _tcA2_body(h_ref, dega_ref, degb_ref, ha_ref, hb_ref, dis_ref):
    deg = dega_ref[...] + degb_ref[...] + 1.0
    dis = lax.rsqrt(deg)
    hs = h_ref[...] * dis
    ha_ref[...] = hs[:, :128]
    hb_ref[...] = hs[:, 128:]
    dis_ref[...] = dis


def _tcA2(h1, dega2, degb2):
    return pl.pallas_call(
        _tcA2_body,
        grid=(N // BLK,),
        in_specs=[
            pl.BlockSpec((BLK, D_HID), lambda i: (i, 0)),
            pl.BlockSpec((BLK, 1), lambda i: (i, 0)),
            pl.BlockSpec((BLK, 1), lambda i: (i, 0)),
        ],
        out_specs=[
            pl.BlockSpec((BLK, 128), lambda i: (i, 0)),
            pl.BlockSpec((BLK, 128), lambda i: (i, 0)),
            pl.BlockSpec((BLK, 1), lambda i: (i, 0)),
        ],
        out_shape=[
            jax.ShapeDtypeStruct((N, 128), jnp.float32),
            jax.ShapeDtypeStruct((N, 128), jnp.float32),
            jax.ShapeDtypeStruct((N, 1), jnp.float32),
        ],
    )(h1, dega2, degb2)


def _tcB_body(pa_ref, pb_ref, dis_ref, w2a_ref, w2b_ref, b2_ref, out_ref):
    d = dis_ref[...]
    ra = jnp.maximum(pa_ref[...] * d, 0.0)
    rb = jnp.maximum(pb_ref[...] * d, 0.0)
    h2 = (lax.dot_general(ra, w2a_ref[...], (((1,), (1,)), ((), ())),
                          precision=_PREC, preferred_element_type=jnp.float32)
          + lax.dot_general(rb, w2b_ref[...], (((1,), (1,)), ((), ())),
                            precision=_PREC, preferred_element_type=jnp.float32)
          + b2_ref[...])
    out_ref[...] = h2 * d


def _tcB(acca, accb, dis, W2a, W2b, b2r):
    return pl.pallas_call(
        _tcB_body,
        grid=(N // BLK,),
        in_specs=[
            pl.BlockSpec((BLK, 128), lambda i: (i, 0)),
            pl.BlockSpec((BLK, 128), lambda i: (i, 0)),
            pl.BlockSpec((BLK, 1), lambda i: (i, 0)),
            pl.BlockSpec((D_OUT, 128), lambda i: (0, 0)),
            pl.BlockSpec((D_OUT, 128), lambda i: (0, 0)),
            pl.BlockSpec((1, D_OUT), lambda i: (0, 0)),
        ],
        out_specs=pl.BlockSpec((BLK, D_OUT), lambda i: (i, 0)),
        out_shape=jax.ShapeDtypeStruct((N, D_OUT), jnp.float32),
    )(acca, accb, dis, W2a, W2b, b2r)


def _tcC_body(a0_ref, a1_ref, h2s_ref, dis_ref, out_ref):
    p = (a0_ref[...] + a1_ref[...] - h2s_ref[...]) * dis_ref[...]
    m = jnp.max(p, axis=1, keepdims=True)
    e = jnp.exp(p - m)
    s = jnp.sum(e, axis=1, keepdims=True)
    out_ref[...] = p - m - jnp.log(s)


def _tcC(a0, a1, h2s, dis):
    return pl.pallas_call(
        _tcC_body,
        grid=(N // BLK,),
        in_specs=[
            pl.BlockSpec((BLK, D_OUT), lambda i: (i, 0)),
            pl.BlockSpec((BLK, D_OUT), lambda i: (i, 0)),
            pl.BlockSpec((BLK, D_OUT), lambda i: (i, 0)),
            pl.BlockSpec((BLK, 1), lambda i: (i, 0)),
        ],
        out_specs=pl.BlockSpec((BLK, D_OUT), lambda i: (i, 0)),
        out_shape=jax.ShapeDtypeStruct((N, D_OUT), jnp.float32),
    )(a0, a1, h2s, dis)


def kernel(x, edge_index, W1, b1, W2, b2):
    row = edge_index[0]
    col = edge_index[1]
    rc2, dega, degb = _deg_call(row, col)
    h1 = _tcA1(x, W1, b1.reshape(1, D_HID))
    ha, hb, dis = _tcA2(h1, dega[:N].reshape(N, 1), degb[:N].reshape(N, 1))
    acca, accb = _prop1_call(ha, hb, rc2)
    h2s = _tcB(acca, accb, dis, W2[:, :128], W2[:, 128:], b2.reshape(1, D_OUT))
    a0, a1 = _prop2_call(h2s, rc2)
    return _tcC(a0, a1, h2s, dis)
